# baseline (speedup 1.0000x reference)
import functools

import jax
import jax.numpy as jnp
import numpy as np
from jax import lax
from jax.experimental import pallas as pl
from jax.experimental.pallas import tpu as pltpu
from jax.experimental.pallas import tpu_sc as plsc

_C = 100000
_B = 1024
_SMOOTH = 0.1

_FILL = float(np.float32(_SMOOTH / (_C - 1)))
_ON = float(np.float32(1.0 - _SMOOTH))
_CONST = _B * ((_C - 1) * _FILL * np.log(_FILL) + _ON * np.log(_ON))
_INV_N = 1.0 / (_B * _C)
_K0 = np.float32(_CONST * _INV_N)
_K1 = np.float32(-_FILL * _INV_N)
_K2 = np.float32(-(_ON - _FILL) * _INV_N)

_NC, _NS, _NL = 2, 16, 16
_NW = _NC * _NS
_RPW = _B // _NW
_GPR = _C // _NL

_TC_COLS = 80000
_TC_ROWS = (_B * _C) // _TC_COLS
_TC_BLK_ROWS = 32
_TC_GRID = _TC_ROWS // _TC_BLK_ROWS


def _sc_gather_body(xf, tgt, out, t_v, idx_v, vals_v, part_v, sem):
    wid = lax.axis_index("s") * _NC + lax.axis_index("c")
    base = wid * _RPW
    pltpu.sync_copy(tgt.at[pl.ds(base, _RPW)], t_v)
    for h in range(_RPW // _NL):
        t = t_v[pl.ds(h * _NL, _NL)]
        rows = lax.iota(jnp.int32, _NL) + (base + h * _NL)
        idx_v[pl.ds(h * _NL, _NL)] = rows * _C + t
    pltpu.async_copy(xf.at[idx_v], vals_v, sem).wait()
    part_v[...] = vals_v[pl.ds(0, _NL)] + vals_v[pl.ds(_NL, _NL)]
    pltpu.sync_copy(part_v, out.at[wid])


@functools.cache
def _sc_gather():
    return functools.partial(
        pl.kernel,
        mesh=plsc.VectorSubcoreMesh(core_axis_name="c", subcore_axis_name="s"),
        out_type=jax.ShapeDtypeStruct((_NW, _NL), jnp.float32),
        scratch_types=[
            pltpu.VMEM((_RPW,), jnp.int32),
            pltpu.VMEM((_RPW,), jnp.int32),
            pltpu.VMEM((_RPW,), jnp.float32),
            pltpu.VMEM((_NL,), jnp.float32),
            pltpu.SemaphoreType.DMA,
        ],
    )(_sc_gather_body)


def _tc_sum_body(x_ref, gp_ref, out_ref, acc_ref):
    i = pl.program_id(0)

    @pl.when(i == 0)
    def _init():
        acc_ref[0, 0] = 0.0
        out_ref[0, 0] = 0.0

    acc_ref[0, 0] += jnp.sum(x_ref[...])

    @pl.when(i == _TC_GRID - 1)
    def _fin():
        g = jnp.sum(gp_ref[...])
        out_ref[0, 0] = _K0 + _K1 * acc_ref[0, 0] + _K2 * g


def kernel(X, target):
    xf = X.reshape(_B * _C)
    gpart = _sc_gather()(xf, target)
    xr = X.reshape(_TC_ROWS, _TC_COLS)
    out = pl.pallas_call(
        _tc_sum_body,
        grid=(_TC_GRID,),
        in_specs=[
            pl.BlockSpec((_TC_BLK_ROWS, _TC_COLS), lambda i: (i, 0)),
            pl.BlockSpec((_NW, _NL), lambda i: (0, 0)),
        ],
        out_specs=pl.BlockSpec(
            (1, 1), lambda i: (0, 0), memory_space=pltpu.SMEM
        ),
        out_shape=jax.ShapeDtypeStruct((1, 1), jnp.float32),
        scratch_shapes=[pltpu.SMEM((1, 1), jnp.float32)],
    )(xr, gpart)
    return out.reshape(())


# speedup vs baseline: 3.2963x; 3.2963x over previous
import functools

import jax
import jax.numpy as jnp
import numpy as np
from jax import lax
from jax.experimental import pallas as pl
from jax.experimental.pallas import tpu as pltpu
from jax.experimental.pallas import tpu_sc as plsc

_C = 100000
_B = 1024
_SMOOTH = 0.1

_FILL = float(np.float32(_SMOOTH / (_C - 1)))
_ON = float(np.float32(1.0 - _SMOOTH))
_CONST = _B * ((_C - 1) * _FILL * np.log(_FILL) + _ON * np.log(_ON))
_INV_N = 1.0 / (_B * _C)
_K0 = np.float32(_CONST * _INV_N)
_K1 = np.float32(-_FILL * _INV_N)
_K2 = np.float32(-(_ON - _FILL) * _INV_N)

_NC, _NS, _NL = 2, 16, 16
_NW = _NC * _NS
_RPW = _B // _NW

_TC_BLK_ROWS = 32
_TC_GRID = _B // _TC_BLK_ROWS


def _sc_gather_body(x, tgt, out, t_v, tiles_v, part_v, sem):
    wid = lax.axis_index("s") * _NC + lax.axis_index("c")
    base = wid * _RPW
    pltpu.sync_copy(tgt.at[pl.ds(base, _RPW)], t_v)
    copies = []
    for h in range(_RPW // _NL):
        tcb = t_v[pl.ds(h * _NL, _NL)] & -128
        for l in range(_NL):
            j = h * _NL + l
            rowb = pl.multiple_of((base + j) & -8, 8)
            colb = pl.multiple_of(tcb[l], 128)
            copies.append(
                pltpu.async_copy(
                    x.at[pl.ds(rowb, 8), pl.ds(colb, 128)],
                    tiles_v.at[j],
                    sem,
                )
            )
    for c in copies:
        c.wait()
    acc = None
    lane = lax.iota(jnp.int32, _NL)
    for h in range(_RPW // _NL):
        tv = t_v[pl.ds(h * _NL, _NL)]
        tgl = tv & (128 - _NL)
        tcol = tv & (_NL - 1)
        for l in range(_NL):
            j = h * _NL + l
            row16 = tiles_v[j, (base + j) % 8, pl.ds(tgl[l], _NL)]
            sel = jnp.where(lane == tcol[l], row16, 0.0)
            acc = sel if acc is None else acc + sel
    part_v[...] = acc
    pltpu.sync_copy(part_v, out.at[wid])


@functools.cache
def _sc_gather():
    return functools.partial(
        pl.kernel,
        mesh=plsc.VectorSubcoreMesh(core_axis_name="c", subcore_axis_name="s"),
        out_type=jax.ShapeDtypeStruct((_NW, _NL), jnp.float32),
        scratch_types=[
            pltpu.VMEM((_RPW,), jnp.int32),
            pltpu.VMEM((_RPW, 8, 128), jnp.float32),
            pltpu.VMEM((_NL,), jnp.float32),
            pltpu.SemaphoreType.DMA,
        ],
    )(_sc_gather_body)


def _tc_sum_body(x_ref, g_ref, out_ref, acc_ref):
    i = pl.program_id(0)

    @pl.when(i == 0)
    def _init():
        acc_ref[0, 0] = 0.0
        out_ref[0, 0] = 0.0

    acc_ref[0, 0] += jnp.sum(x_ref[...])

    @pl.when(i == _TC_GRID - 1)
    def _fin():
        g = jnp.sum(g_ref[...])
        out_ref[0, 0] = _K0 + _K1 * acc_ref[0, 0] + _K2 * g


def kernel(X, target):
    gpart = _sc_gather()(X, target)
    out = pl.pallas_call(
        _tc_sum_body,
        grid=(_TC_GRID,),
        in_specs=[
            pl.BlockSpec((_TC_BLK_ROWS, _C), lambda i: (i, 0)),
            pl.BlockSpec((_NW, _NL), lambda i: (0, 0)),
        ],
        out_specs=pl.BlockSpec(
            (1, 1), lambda i: (0, 0), memory_space=pltpu.SMEM
        ),
        out_shape=jax.ShapeDtypeStruct((1, 1), jnp.float32),
        scratch_shapes=[pltpu.SMEM((1, 1), jnp.float32)],
    )(X, gpart)
    return out.reshape(())


# speedup vs baseline: 3.3203x; 1.0073x over previous
import functools

import jax
import jax.numpy as jnp
import numpy as np
from jax import lax
from jax.experimental import pallas as pl
from jax.experimental.pallas import tpu as pltpu
from jax.experimental.pallas import tpu_sc as plsc

_C = 100000
_B = 1024
_SMOOTH = 0.1

_FILL = float(np.float32(_SMOOTH / (_C - 1)))
_ON = float(np.float32(1.0 - _SMOOTH))
_CONST = _B * ((_C - 1) * _FILL * np.log(_FILL) + _ON * np.log(_ON))
_INV_N = 1.0 / (_B * _C)
_K0 = np.float32(_CONST * _INV_N)
_K1 = np.float32(-_FILL * _INV_N)
_K2 = np.float32(-(_ON - _FILL) * _INV_N)

_NC, _NS, _NL = 2, 16, 16
_NW = _NC * _NS
_RPW = _B // _NW

_TC_BLK_ROWS = 64
_TC_GRID = _B // _TC_BLK_ROWS


def _sc_gather_body(x, tgt, out, t_v, tiles_v, part_v, sem):
    wid = lax.axis_index("s") * _NC + lax.axis_index("c")
    base = wid * _RPW
    pltpu.sync_copy(tgt.at[pl.ds(base, _RPW)], t_v)
    copies = []
    for h in range(_RPW // _NL):
        tcb = t_v[pl.ds(h * _NL, _NL)] & -128
        for l in range(_NL):
            j = h * _NL + l
            rowb = pl.multiple_of((base + j) & -8, 8)
            colb = pl.multiple_of(tcb[l], 128)
            copies.append(
                pltpu.async_copy(
                    x.at[pl.ds(rowb, 8), pl.ds(colb, 128)],
                    tiles_v.at[j],
                    sem,
                )
            )
    for c in copies:
        c.wait()
    acc = None
    lane = lax.iota(jnp.int32, _NL)
    for h in range(_RPW // _NL):
        tv = t_v[pl.ds(h * _NL, _NL)]
        tgl = tv & (128 - _NL)
        tcol = tv & (_NL - 1)
        for l in range(_NL):
            j = h * _NL + l
            row16 = tiles_v[j, (base + j) % 8, pl.ds(tgl[l], _NL)]
            sel = jnp.where(lane == tcol[l], row16, 0.0)
            acc = sel if acc is None else acc + sel
    part_v[...] = acc
    pltpu.sync_copy(part_v, out.at[wid])


@functools.cache
def _sc_gather():
    return functools.partial(
        pl.kernel,
        mesh=plsc.VectorSubcoreMesh(core_axis_name="c", subcore_axis_name="s"),
        out_type=jax.ShapeDtypeStruct((_NW, _NL), jnp.float32),
        scratch_types=[
            pltpu.VMEM((_RPW,), jnp.int32),
            pltpu.VMEM((_RPW, 8, 128), jnp.float32),
            pltpu.VMEM((_NL,), jnp.float32),
            pltpu.SemaphoreType.DMA,
        ],
    )(_sc_gather_body)


def _tc_sum_body(x_ref, g_ref, out_ref, acc_ref):
    i = pl.program_id(0)

    @pl.when(i == 0)
    def _init():
        acc_ref[0, 0] = 0.0
        out_ref[0, 0] = 0.0

    acc_ref[0, 0] += jnp.sum(x_ref[...])

    @pl.when(i == _TC_GRID - 1)
    def _fin():
        g = jnp.sum(g_ref[...])
        out_ref[0, 0] = _K0 + _K1 * acc_ref[0, 0] + _K2 * g


def kernel(X, target):
    gpart = _sc_gather()(X, target)
    out = pl.pallas_call(
        _tc_sum_body,
        grid=(_TC_GRID,),
        in_specs=[
            pl.BlockSpec((_TC_BLK_ROWS, _C), lambda i: (i, 0)),
            pl.BlockSpec((_NW, _NL), lambda i: (0, 0)),
        ],
        out_specs=pl.BlockSpec(
            (1, 1), lambda i: (0, 0), memory_space=pltpu.SMEM
        ),
        out_shape=jax.ShapeDtypeStruct((1, 1), jnp.float32),
        scratch_shapes=[pltpu.SMEM((1, 1), jnp.float32)],
    )(X, gpart)
    return out.reshape(())
